# baseline (speedup 1.0000x reference)
import functools

import jax
import jax.numpy as jnp
from jax import lax
from jax.experimental import pallas as pl
from jax.experimental.pallas import tpu as pltpu
from jax.experimental.pallas import tpu_sc as plsc

B = 1024
D = 512
R = 512
M = 65536
K = 512
RECENCY_WEIGHT = 0.3
NSUB = 16
CHUNK = M // NSUB
ROWS_PER_SUB = K // NSUB


_BM = 4096
_NBLK = M // _BM


def _scores_body(q_ref, qp_ref, ms_ref, s_ref):
    i = pl.program_id(0)
    q = q_ref[0:1, :]
    blk = ms_ref[...]
    qn = jnp.sqrt(jnp.sum(q * q))
    dots_full = jnp.dot(blk.astype(jnp.bfloat16), qp_ref[...],
                        preferred_element_type=jnp.float32)
    dots = dots_full[:, 0:1]
    n2 = jnp.sum(blk * blk, axis=1, keepdims=True)
    denom = jnp.maximum(jnp.sqrt(n2) * qn, 1e-8)
    sims = dots / denom
    row = (lax.broadcasted_iota(jnp.int32, (_BM, 1), 0)
           + i * _BM).astype(jnp.float32)
    rec = row * (1.0 / (M - 1))
    s_ref[...] = (1.0 - RECENCY_WEIGHT) * sims + RECENCY_WEIGHT * rec


def _scores_call(query, qpad, mem_states):
    return pl.pallas_call(
        _scores_body,
        grid=(_NBLK,),
        in_specs=[
            pl.BlockSpec((8, D), lambda i: (0, 0)),
            pl.BlockSpec((D, 128), lambda i: (0, 0)),
            pl.BlockSpec((_BM, D), lambda i: (i, 0)),
        ],
        out_specs=pl.BlockSpec((_BM, 1), lambda i: (i, 0)),
        out_shape=jax.ShapeDtypeStruct((M, 1), jnp.float32),
    )(query, qpad, mem_states)


def _thresh_body(s_ref, t_ref, c_ref):
    s = s_ref[...]
    v = lax.bitcast_convert_type(s, jnp.int32)
    u = v ^ ((v >> 31) & jnp.int32(0x7FFFFFFF))
    n_nonneg = jnp.sum((u >= 0).astype(jnp.int32))
    t0 = jnp.where(n_nonneg >= K, jnp.int32(0), jnp.int32(-2147483648))

    def step(j, t):
        b = 30 - j
        t_try = t | (jnp.int32(1) << b)
        cnt = jnp.sum((u >= t_try).astype(jnp.int32))
        return jnp.where(cnt >= K, t_try, t)

    t_key = lax.fori_loop(0, 31, step, t0)
    cnt_gt = jnp.sum((u > t_key).astype(jnp.int32))
    vt = t_key ^ ((t_key >> 31) & jnp.int32(0x7FFFFFFF))
    t_ref[0, 0] = lax.bitcast_convert_type(vt, jnp.float32)
    c_ref[0, 0] = cnt_gt.astype(jnp.float32)


def _thresh_call(scores2d):
    return pl.pallas_call(
        _thresh_body,
        in_specs=[pl.BlockSpec((M // 128, 128), lambda: (0, 0))],
        out_specs=[
            pl.BlockSpec(memory_space=pltpu.SMEM),
            pl.BlockSpec(memory_space=pltpu.SMEM),
        ],
        out_shape=[
            jax.ShapeDtypeStruct((1, 1), jnp.float32),
            jax.ShapeDtypeStruct((1, 1), jnp.float32),
        ],
    )(scores2d)


def _sc_body(scores_hbm, meta_hbm, ms_hbm, rew_hbm,
             eps_out, rew_out, sh_gt, sh_eq, sh_sel,
             chunk_v, meta_v, stage_gt_v, stage_eq_v, cgt_m, ceq_m,
             sel_v, selmat_v, pvec_v, myidx_v, rows_v, rewtab_v, rewbuf_v,
             sem):
    sid = lax.axis_index("s")
    lane = lax.iota(jnp.int32, 16)
    one16 = jnp.ones((16,), jnp.int32)
    zero16 = jnp.zeros((16,), jnp.int32)

    base = sid * CHUNK
    pltpu.sync_copy(scores_hbm.at[pl.ds(base, CHUNK)], chunk_v)
    pltpu.sync_copy(meta_hbm, meta_v)
    t_vec = meta_v[0, :]
    need = K - meta_v[1, :].astype(jnp.int32)

    def count_step(j, carry):
        cgt, ceq = carry
        sv = chunk_v[pl.ds(j * 16, 16)]
        cgt = cgt + plsc.all_reduce_population_count(sv > t_vec)
        ceq = ceq + plsc.all_reduce_population_count(sv == t_vec)
        return cgt, ceq

    cgt, ceq = lax.fori_loop(0, CHUNK // 16, count_step, (zero16, zero16))

    stage_gt_v[...] = cgt
    stage_eq_v[...] = ceq
    pltpu.sync_copy(stage_gt_v, sh_gt.at[sid])
    pltpu.sync_copy(stage_eq_v, sh_eq.at[sid])

    @pl.when(sid == 0)
    def _():
        pltpu.sync_copy(rew_hbm, rewtab_v)

    plsc.subcore_barrier()

    pltpu.sync_copy(sh_gt, cgt_m)
    pltpu.sync_copy(sh_eq, ceq_m)
    pre_eq = zero16
    my_pre_eq = zero16
    run_p = zero16
    p_list = []
    pvec = zero16
    for w in range(NSUB):
        gt_w = cgt_m[w]
        eq_w = ceq_m[w]
        take_w = gt_w + jnp.clip(need - pre_eq, 0, eq_w)
        p_list.append(run_p)
        pvec = jnp.where(lane == w, run_p, pvec)
        my_pre_eq = jnp.where(jnp.int32(w) < sid, pre_eq + eq_w, my_pre_eq)
        pre_eq = pre_eq + eq_w
        run_p = run_p + take_w
    pvec_v[...] = pvec

    def compact_step(j, carry):
        nloc, beq = carry
        sv = chunk_v[pl.ds(j * 16, 16)]
        m_gt = sv > t_vec
        m_eq = sv == t_vec
        eq_rank = beq + plsc.cumsum(m_eq.astype(jnp.int32)) - 1
        m_take = jnp.logical_or(m_gt, jnp.logical_and(m_eq, eq_rank < need))
        ranks = plsc.cumsum(m_take.astype(jnp.int32)) - 1
        pos = jnp.where(m_take, nloc + ranks, zero16)
        gidx = base + j * 16 + lane
        plsc.store_scatter(sel_v, [pos], gidx, mask=m_take)
        return (nloc + plsc.all_reduce_population_count(m_take),
                beq + plsc.all_reduce_population_count(m_eq))

    lax.fori_loop(0, CHUNK // 16, compact_step, (zero16, my_pre_eq))
    pltpu.sync_copy(sel_v, sh_sel.at[sid])
    plsc.subcore_barrier()

    pltpu.sync_copy(sh_sel, selmat_v)

    def ranks_to_idx(rnk):
        src = zero16
        for w in range(1, NSUB):
            src = src + jnp.where(rnk >= p_list[w], one16, zero16)
        pbase = plsc.load_gather(pvec_v, [src])
        loc = jnp.clip(rnk - pbase, 0, K - 1)
        return jnp.clip(plsc.load_gather(selmat_v, [src, loc]), 0, M - 1)

    for j in range(ROWS_PER_SUB // 16):
        rnk = sid * ROWS_PER_SUB + j * 16 + lane
        myidx_v[pl.ds(j * 16, 16)] = ranks_to_idx(rnk)

    pltpu.async_copy(ms_hbm.at[myidx_v], rows_v, sem).wait()
    pltpu.sync_copy(rows_v, eps_out.at[pl.ds(sid * ROWS_PER_SUB, ROWS_PER_SUB)])

    @pl.when(sid == 0)
    def _():
        for j in range(K // 16):
            idx16 = ranks_to_idx(j * 16 + lane)
            rewbuf_v[pl.ds(j * 16, 16)] = plsc.load_gather(rewtab_v, [idx16])
        pltpu.sync_copy(rewbuf_v, rew_out)


def _sc_call(scores_flat, meta, mem_states, mem_rewards):
    mesh = plsc.VectorSubcoreMesh(core_axis_name="c", subcore_axis_name="s",
                                  num_cores=1)
    fn = pl.kernel(
        _sc_body,
        mesh=mesh,
        compiler_params=pltpu.CompilerParams(needs_layout_passes=False),
        out_type=[
            jax.ShapeDtypeStruct((K, D), jnp.float32),
            jax.ShapeDtypeStruct((K,), jnp.float32),
            jax.ShapeDtypeStruct((NSUB, 16), jnp.int32),
            jax.ShapeDtypeStruct((NSUB, 16), jnp.int32),
            jax.ShapeDtypeStruct((NSUB, K), jnp.int32),
        ],
        scratch_types=[
            pltpu.VMEM((CHUNK,), jnp.float32),
            pltpu.VMEM((2, 16), jnp.float32),
            pltpu.VMEM((16,), jnp.int32),
            pltpu.VMEM((16,), jnp.int32),
            pltpu.VMEM((NSUB, 16), jnp.int32),
            pltpu.VMEM((NSUB, 16), jnp.int32),
            pltpu.VMEM((K,), jnp.int32),
            pltpu.VMEM((NSUB, K), jnp.int32),
            pltpu.VMEM((16,), jnp.int32),
            pltpu.VMEM((ROWS_PER_SUB,), jnp.int32),
            pltpu.VMEM((ROWS_PER_SUB, D), jnp.float32),
            pltpu.VMEM((M,), jnp.float32),
            pltpu.VMEM((K,), jnp.float32),
            pltpu.SemaphoreType.DMA,
        ],
    )
    ep, er, _, _, _ = fn(scores_flat, meta, mem_states, mem_rewards)
    return ep, er


def _attn_body(q_ref, ep_ref, er_ref, wq_ref, bq_ref, wk_ref, bk_ref,
               wvs_ref, wvr_ref, bv_ref, wo_ref, bo_ref, g_ref, be_ref,
               out_ref):
    query = q_ref[...].astype(jnp.bfloat16)
    ep = ep_ref[...].astype(jnp.bfloat16)
    er = er_ref[...]
    Q = (jnp.dot(query, wq_ref[...].astype(jnp.bfloat16),
                 preferred_element_type=jnp.float32) + bq_ref[...])
    Kp = (jnp.dot(ep, wk_ref[...].astype(jnp.bfloat16),
                  preferred_element_type=jnp.float32) + bk_ref[...])
    V = (jnp.dot(ep, wvs_ref[...].astype(jnp.bfloat16),
                 preferred_element_type=jnp.float32)
         + er * wvr_ref[...] + bv_ref[...])
    att = lax.dot_general(Q.astype(jnp.bfloat16), Kp.astype(jnp.bfloat16),
                          (((1,), (1,)), ((), ())),
                          preferred_element_type=jnp.float32)
    att = att / jnp.sqrt(jnp.float32(R))
    mx = jnp.max(att, axis=1, keepdims=True)
    e = jnp.exp(att - mx)
    w = e / jnp.sum(e, axis=1, keepdims=True)
    ctx = jnp.dot(w.astype(jnp.bfloat16), V.astype(jnp.bfloat16),
                  preferred_element_type=jnp.float32)
    h = (jnp.dot(ctx.astype(jnp.bfloat16), wo_ref[...].astype(jnp.bfloat16),
                 preferred_element_type=jnp.float32) + bo_ref[...])
    mu = jnp.mean(h, axis=1, keepdims=True)
    var = jnp.mean((h - mu) * (h - mu), axis=1, keepdims=True)
    hn = (h - mu) / jnp.sqrt(var + 1e-5) * g_ref[...] + be_ref[...]
    out_ref[...] = 0.5 * hn * (1.0 + lax.erf(hn * 0.7071067811865476))


def _attn_call(query, ep, er2, Wq, bq2, Wk, bk2, Wvs, wvr2, bv2, Wo, bo2, g2, be2):
    return pl.pallas_call(
        _attn_body,
        out_shape=jax.ShapeDtypeStruct((B, D), jnp.float32),
    )(query, ep, er2, Wq, bq2, Wk, bk2, Wvs, wvr2, bv2, Wo, bo2, g2, be2)


def kernel(query, k, mem_states, mem_rewards, mem_timestamps,
           Wq, bq, Wk, bk, Wv, bv, Wo, bo, ln_gamma, ln_beta):
    q0b = query[0].astype(jnp.bfloat16)
    qpad = jnp.zeros((D, 128), jnp.bfloat16).at[:, 0].set(q0b)
    scores = _scores_call(query, qpad, mem_states)
    t_f, cnt_gt = _thresh_call(scores.reshape(M // 128, 128))
    meta = jnp.concatenate([
        jnp.broadcast_to(t_f.reshape(1, 1), (1, 16)),
        jnp.broadcast_to(cnt_gt.reshape(1, 1), (1, 16)),
    ], axis=0)
    ep, er = _sc_call(scores.reshape(M), meta, mem_states, mem_rewards)
    out = _attn_call(
        query, ep, er.reshape(K, 1),
        Wq, bq.reshape(1, R), Wk, bk.reshape(1, R),
        Wv[:D], Wv[D].reshape(1, R), bv.reshape(1, R),
        Wo, bo.reshape(1, D), ln_gamma.reshape(1, D), ln_beta.reshape(1, D),
    )
    return out


# speedup vs baseline: 1.2142x; 1.2142x over previous
import functools

import jax
import jax.numpy as jnp
from jax import lax
from jax.experimental import pallas as pl
from jax.experimental.pallas import tpu as pltpu
from jax.experimental.pallas import tpu_sc as plsc

B = 1024
D = 512
R = 512
M = 65536
K = 512
RECENCY_WEIGHT = 0.3
NSUB = 16
CHUNK = M // NSUB
ROWS_PER_SUB = K // NSUB


_BM = 4096
_NBLK = M // _BM


def _scores_body(q_ref, qp_ref, ms_ref, s_ref):
    i = pl.program_id(0)
    q = q_ref[0:1, :]
    blk = ms_ref[...]
    qn = jnp.sqrt(jnp.sum(q * q))
    dots_full = jnp.dot(blk.astype(jnp.bfloat16), qp_ref[...],
                        preferred_element_type=jnp.float32)
    dots = dots_full[:, 0:1]
    n2 = jnp.sum(blk * blk, axis=1, keepdims=True)
    denom = jnp.maximum(jnp.sqrt(n2) * qn, 1e-8)
    sims = dots / denom
    row = (lax.broadcasted_iota(jnp.int32, (_BM, 1), 0)
           + i * _BM).astype(jnp.float32)
    rec = row * (1.0 / (M - 1))
    s = (1.0 - RECENCY_WEIGHT) * sims + RECENCY_WEIGHT * rec
    s_ref[...] = s.reshape(_BM // 128, 128)


def _scores_call(query, qpad, mem_states):
    return pl.pallas_call(
        _scores_body,
        grid=(_NBLK,),
        in_specs=[
            pl.BlockSpec((8, D), lambda i: (0, 0)),
            pl.BlockSpec((D, 128), lambda i: (0, 0)),
            pl.BlockSpec((_BM, D), lambda i: (i, 0)),
        ],
        out_specs=pl.BlockSpec((_BM // 128, 128), lambda i: (i, 0)),
        out_shape=jax.ShapeDtypeStruct((M // 128, 128), jnp.float32),
    )(query, qpad, mem_states)


def _thresh_body(s_ref, t_ref, c_ref):
    s = s_ref[...]
    v = lax.bitcast_convert_type(s, jnp.int32)
    u = v ^ ((v >> 31) & jnp.int32(0x7FFFFFFF))
    n_nonneg = jnp.sum((u >= 0).astype(jnp.int32))
    t0 = jnp.where(n_nonneg >= K, jnp.int32(0), jnp.int32(-2147483648))

    def step(j, t):
        b = 30 - j
        t_try = t | (jnp.int32(1) << b)
        cnt = jnp.sum((u >= t_try).astype(jnp.int32))
        return jnp.where(cnt >= K, t_try, t)

    t_key = lax.fori_loop(0, 31, step, t0)
    cnt_gt = jnp.sum((u > t_key).astype(jnp.int32))
    vt = t_key ^ ((t_key >> 31) & jnp.int32(0x7FFFFFFF))
    t_ref[0, 0] = lax.bitcast_convert_type(vt, jnp.float32)
    c_ref[0, 0] = cnt_gt.astype(jnp.float32)


def _thresh_call(scores2d):
    return pl.pallas_call(
        _thresh_body,
        in_specs=[pl.BlockSpec((M // 128, 128), lambda: (0, 0))],
        out_specs=[
            pl.BlockSpec(memory_space=pltpu.SMEM),
            pl.BlockSpec(memory_space=pltpu.SMEM),
        ],
        out_shape=[
            jax.ShapeDtypeStruct((1, 1), jnp.float32),
            jax.ShapeDtypeStruct((1, 1), jnp.float32),
        ],
    )(scores2d)


def _sc_body(scores_hbm, meta_hbm, ms_hbm, rew_hbm,
             eps_out, rew_out, sh_gt, sh_eq, sh_sel,
             chunk_v, meta_v, stage_gt_v, stage_eq_v, cgt_m, ceq_m,
             sel_v, selmat_v, pvec_v, myidx_v, rows_v, rewtab_v, rewbuf_v,
             sem):
    sid = lax.axis_index("s")
    lane = lax.iota(jnp.int32, 16)
    one16 = jnp.ones((16,), jnp.int32)
    zero16 = jnp.zeros((16,), jnp.int32)

    base = sid * CHUNK
    pltpu.sync_copy(scores_hbm.at[pl.ds(base, CHUNK)], chunk_v)
    pltpu.sync_copy(meta_hbm, meta_v)
    t_vec = meta_v[0, :]
    need = K - meta_v[1, :].astype(jnp.int32)

    def count_step(j, carry):
        cgt, ceq = carry
        sv = chunk_v[pl.ds(j * 16, 16)]
        cgt = cgt + plsc.all_reduce_population_count(sv > t_vec)
        ceq = ceq + plsc.all_reduce_population_count(sv == t_vec)
        return cgt, ceq

    cgt, ceq = lax.fori_loop(0, CHUNK // 16, count_step, (zero16, zero16))

    stage_gt_v[...] = cgt
    stage_eq_v[...] = ceq
    pltpu.sync_copy(stage_gt_v, sh_gt.at[sid])
    pltpu.sync_copy(stage_eq_v, sh_eq.at[sid])

    @pl.when(sid == 0)
    def _():
        pltpu.sync_copy(rew_hbm, rewtab_v)

    plsc.subcore_barrier()

    pltpu.sync_copy(sh_gt, cgt_m)
    pltpu.sync_copy(sh_eq, ceq_m)
    pre_eq = zero16
    my_pre_eq = zero16
    run_p = zero16
    p_list = []
    pvec = zero16
    for w in range(NSUB):
        gt_w = cgt_m[w]
        eq_w = ceq_m[w]
        take_w = gt_w + jnp.clip(need - pre_eq, 0, eq_w)
        p_list.append(run_p)
        pvec = jnp.where(lane == w, run_p, pvec)
        my_pre_eq = jnp.where(jnp.int32(w) < sid, pre_eq + eq_w, my_pre_eq)
        pre_eq = pre_eq + eq_w
        run_p = run_p + take_w
    pvec_v[...] = pvec

    def compact_step(j, carry):
        nloc, beq = carry
        sv = chunk_v[pl.ds(j * 16, 16)]
        m_gt = sv > t_vec
        m_eq = sv == t_vec
        eq_rank = beq + plsc.cumsum(m_eq.astype(jnp.int32)) - 1
        m_take = jnp.logical_or(m_gt, jnp.logical_and(m_eq, eq_rank < need))
        ranks = plsc.cumsum(m_take.astype(jnp.int32)) - 1
        pos = jnp.where(m_take, nloc + ranks, zero16)
        gidx = base + j * 16 + lane
        plsc.store_scatter(sel_v, [pos], gidx, mask=m_take)
        return (nloc + plsc.all_reduce_population_count(m_take),
                beq + plsc.all_reduce_population_count(m_eq))

    lax.fori_loop(0, CHUNK // 16, compact_step, (zero16, my_pre_eq))
    pltpu.sync_copy(sel_v, sh_sel.at[sid])
    plsc.subcore_barrier()

    pltpu.sync_copy(sh_sel, selmat_v)

    def ranks_to_idx(rnk):
        src = zero16
        for w in range(1, NSUB):
            src = src + jnp.where(rnk >= p_list[w], one16, zero16)
        pbase = plsc.load_gather(pvec_v, [src])
        loc = jnp.clip(rnk - pbase, 0, K - 1)
        return jnp.clip(plsc.load_gather(selmat_v, [src, loc]), 0, M - 1)

    for j in range(ROWS_PER_SUB // 16):
        rnk = sid * ROWS_PER_SUB + j * 16 + lane
        myidx_v[pl.ds(j * 16, 16)] = ranks_to_idx(rnk)

    pltpu.async_copy(ms_hbm.at[myidx_v], rows_v, sem).wait()
    pltpu.sync_copy(rows_v, eps_out.at[pl.ds(sid * ROWS_PER_SUB, ROWS_PER_SUB)])

    @pl.when(sid == 0)
    def _():
        for j in range(K // 16):
            idx16 = ranks_to_idx(j * 16 + lane)
            rewbuf_v[pl.ds(j * 16, 16)] = plsc.load_gather(rewtab_v, [idx16])
        pltpu.sync_copy(rewbuf_v, rew_out)


def _sc_call(scores_flat, meta, mem_states, mem_rewards):
    mesh = plsc.VectorSubcoreMesh(core_axis_name="c", subcore_axis_name="s",
                                  num_cores=1)
    fn = pl.kernel(
        _sc_body,
        mesh=mesh,
        compiler_params=pltpu.CompilerParams(needs_layout_passes=False),
        out_type=[
            jax.ShapeDtypeStruct((K, D), jnp.float32),
            jax.ShapeDtypeStruct((K,), jnp.float32),
            jax.ShapeDtypeStruct((NSUB, 16), jnp.int32),
            jax.ShapeDtypeStruct((NSUB, 16), jnp.int32),
            jax.ShapeDtypeStruct((NSUB, K), jnp.int32),
        ],
        scratch_types=[
            pltpu.VMEM((CHUNK,), jnp.float32),
            pltpu.VMEM((2, 16), jnp.float32),
            pltpu.VMEM((16,), jnp.int32),
            pltpu.VMEM((16,), jnp.int32),
            pltpu.VMEM((NSUB, 16), jnp.int32),
            pltpu.VMEM((NSUB, 16), jnp.int32),
            pltpu.VMEM((K,), jnp.int32),
            pltpu.VMEM((NSUB, K), jnp.int32),
            pltpu.VMEM((16,), jnp.int32),
            pltpu.VMEM((ROWS_PER_SUB,), jnp.int32),
            pltpu.VMEM((ROWS_PER_SUB, D), jnp.float32),
            pltpu.VMEM((M,), jnp.float32),
            pltpu.VMEM((K,), jnp.float32),
            pltpu.SemaphoreType.DMA,
        ],
    )
    ep, er, _, _, _ = fn(scores_flat, meta, mem_states, mem_rewards)
    return ep, er


def _attn_body(q_ref, ep_ref, er_ref, wq_ref, bq_ref, wk_ref, bk_ref,
               wvs_ref, wvr_ref, bv_ref, wo_ref, bo_ref, g_ref, be_ref,
               out_ref):
    query = q_ref[...].astype(jnp.bfloat16)
    ep = ep_ref[...].astype(jnp.bfloat16)
    er = er_ref[...]
    Q = (jnp.dot(query, wq_ref[...].astype(jnp.bfloat16),
                 preferred_element_type=jnp.float32) + bq_ref[...])
    Kp = (jnp.dot(ep, wk_ref[...].astype(jnp.bfloat16),
                  preferred_element_type=jnp.float32) + bk_ref[...])
    V = (jnp.dot(ep, wvs_ref[...].astype(jnp.bfloat16),
                 preferred_element_type=jnp.float32)
         + er * wvr_ref[...] + bv_ref[...])
    att = lax.dot_general(Q.astype(jnp.bfloat16), Kp.astype(jnp.bfloat16),
                          (((1,), (1,)), ((), ())),
                          preferred_element_type=jnp.float32)
    att = att / jnp.sqrt(jnp.float32(R))
    mx = jnp.max(att, axis=1, keepdims=True)
    e = jnp.exp(att - mx)
    w = e / jnp.sum(e, axis=1, keepdims=True)
    ctx = jnp.dot(w.astype(jnp.bfloat16), V.astype(jnp.bfloat16),
                  preferred_element_type=jnp.float32)
    h = (jnp.dot(ctx.astype(jnp.bfloat16), wo_ref[...].astype(jnp.bfloat16),
                 preferred_element_type=jnp.float32) + bo_ref[...])
    mu = jnp.mean(h, axis=1, keepdims=True)
    var = jnp.mean((h - mu) * (h - mu), axis=1, keepdims=True)
    hn = (h - mu) / jnp.sqrt(var + 1e-5) * g_ref[...] + be_ref[...]
    out_ref[...] = 0.5 * hn * (1.0 + lax.erf(hn * 0.7071067811865476))


def _attn_call(query, ep, er2, Wq, bq2, Wk, bk2, Wvs, wvr2, bv2, Wo, bo2, g2, be2):
    return pl.pallas_call(
        _attn_body,
        out_shape=jax.ShapeDtypeStruct((B, D), jnp.float32),
    )(query, ep, er2, Wq, bq2, Wk, bk2, Wvs, wvr2, bv2, Wo, bo2, g2, be2)


def kernel(query, k, mem_states, mem_rewards, mem_timestamps,
           Wq, bq, Wk, bk, Wv, bv, Wo, bo, ln_gamma, ln_beta):
    q0b = query[0].astype(jnp.bfloat16)
    qpad = jnp.zeros((D, 128), jnp.bfloat16).at[:, 0].set(q0b)
    scores = _scores_call(query, qpad, mem_states)
    t_f, cnt_gt = _thresh_call(scores)
    meta = jnp.concatenate([
        jnp.broadcast_to(t_f.reshape(1, 1), (1, 16)),
        jnp.broadcast_to(cnt_gt.reshape(1, 1), (1, 16)),
    ], axis=0)
    ep, er = _sc_call(scores.reshape(M), meta, mem_states, mem_rewards)
    out = _attn_call(
        query, ep, er.reshape(K, 1),
        Wq, bq.reshape(1, R), Wk, bk.reshape(1, R),
        Wv[:D], Wv[D].reshape(1, R), bv.reshape(1, R),
        Wo, bo.reshape(1, D), ln_gamma.reshape(1, D), ln_beta.reshape(1, D),
    )
    return out
